# Initial kernel scaffold; baseline (speedup 1.0000x reference)
import jax
import jax.numpy as jnp
from jax.experimental import pallas as pl


def kernel(x, proj_w, proj_b, out_w, out_b, alpha, beta):
    raise NotImplementedError("write your pallas kernel here")


# speedup vs baseline: 2.2144x; 2.2144x over previous
import numpy as np
import jax
import jax.numpy as jnp
from jax.experimental import pallas as pl

DIM = 384
NHEADS = 6
HDIM = DIM // NHEADS
NANCH = 8
A = NANCH * NANCH
PDIM = 2 * DIM

_PERM = np.concatenate(
    [np.arange(HDIM) + h * 2 * HDIM for h in range(NHEADS)]
    + [np.arange(HDIM) + h * 2 * HDIM + HDIM for h in range(NHEADS)]
)


def _cluster_kernel(x_ref, pw_ref, pb_ref, ow_ref, ob_ref, al_ref, be_ref,
                    out_ref):
    f32 = jnp.float32
    bf16 = jnp.bfloat16
    hw = x_ref.shape[1]

    def mm(a, b, dims):
        return jax.lax.dot_general(a.astype(bf16), b.astype(bf16),
                                   (dims, ((), ())),
                                   preferred_element_type=f32)

    xb = x_ref[0]
    xp = mm(xb, pw_ref[...], ((1,), (1,))) + pb_ref[...]

    m1 = jnp.max(xp.reshape(hw // 4, 4, PDIM), axis=1)
    m2 = jnp.max(m1.reshape(NANCH, 4, NANCH, PDIM), axis=1)
    pooled = m2.reshape(A, PDIM)

    P0, V0 = xp[:, :DIM], xp[:, DIM:]
    P1, V1 = pooled[:, :DIM], pooled[:, DIM:]

    p0n, p1n = [], []
    for h in range(NHEADS):
        s = slice(h * HDIM, (h + 1) * HDIM)
        p0h = P0[:, s]
        n0 = jnp.sqrt(jnp.sum(p0h * p0h, axis=1, keepdims=True))
        p0n.append(p0h / jnp.maximum(n0, 1e-12))
        p1h = P1[:, s]
        n1 = jnp.sqrt(jnp.sum(p1h * p1h, axis=1, keepdims=True))
        p1n.append(p1h / jnp.maximum(n1, 1e-12))
    P0n = jnp.concatenate(p0n, axis=1)
    P1n = jnp.concatenate(p1n, axis=1)

    ri = jax.lax.broadcasted_iota(jnp.int32, (DIM, DIM), 0) // HDIM
    ci = jax.lax.broadcasted_iota(jnp.int32, (DIM, DIM), 1) // HDIM
    bdmask = ri == ci

    P1BD = jnp.where(bdmask, jnp.concatenate([P1n.T] * NHEADS, axis=1), 0.0)
    sim = mm(P0n, P1BD, ((1,), (0,)))
    sim = jax.nn.sigmoid(al_ref[0, 0] * sim + be_ref[0, 0])

    sparts = []
    for h in range(NHEADS):
        sh = sim[:, h * A:(h + 1) * A]
        mx = jnp.max(sh, axis=1, keepdims=True)
        lane = jax.lax.broadcasted_iota(jnp.int32, sh.shape, 1)
        first = jnp.min(jnp.where(sh == mx, lane, A), axis=1, keepdims=True)
        sparts.append(jnp.where(lane == first, sh, 0.0))
    S = jnp.concatenate(sparts, axis=1)

    denom = jnp.sum(S, axis=0, keepdims=True) + 1.0
    Sd = S / denom

    M1 = mm(S, V0, ((0,), (0,)))
    MSG = jnp.where(bdmask, M1 + jnp.concatenate([V1] * NHEADS, axis=0), 0.0)

    out_tok = mm(Sd, MSG, ((1,), (0,)))
    y = mm(out_tok, ow_ref[...], ((1,), (1,))) + ob_ref[...]
    out_ref[0] = y


def kernel(x, proj_w, proj_b, out_w, out_b, alpha, beta):
    n, h, w, _ = x.shape
    hw = h * w
    pw = proj_w[_PERM]
    pb = proj_b[_PERM].reshape(1, PDIM)
    out = pl.pallas_call(
        _cluster_kernel,
        grid=(n,),
        in_specs=[
            pl.BlockSpec((1, hw, DIM), lambda i: (i, 0, 0)),
            pl.BlockSpec((PDIM, DIM), lambda i: (0, 0)),
            pl.BlockSpec((1, PDIM), lambda i: (0, 0)),
            pl.BlockSpec((DIM, DIM), lambda i: (0, 0)),
            pl.BlockSpec((1, DIM), lambda i: (0, 0)),
            pl.BlockSpec((1, 1), lambda i: (0, 0)),
            pl.BlockSpec((1, 1), lambda i: (0, 0)),
        ],
        out_specs=pl.BlockSpec((1, hw, DIM), lambda i: (i, 0, 0)),
        out_shape=jax.ShapeDtypeStruct((n, hw, DIM), jnp.float32),
    )(x.reshape(n, hw, DIM), pw, pb, out_w, out_b.reshape(1, DIM),
      alpha.reshape(1, 1), beta.reshape(1, 1))
    return out.reshape(n, h, w, DIM)


# speedup vs baseline: 3.6350x; 1.6415x over previous
import numpy as np
import jax
import jax.numpy as jnp
from jax.experimental import pallas as pl

DIM = 384
NHEADS = 6
HDIM = DIM // NHEADS
NANCH = 8
A = NANCH * NANCH
PDIM = 2 * DIM

_PERM = np.concatenate(
    [np.arange(HDIM) + h * 2 * HDIM for h in range(NHEADS)]
    + [np.arange(HDIM) + h * 2 * HDIM + HDIM for h in range(NHEADS)]
)


def _cluster_kernel(x_ref, pw_ref, pb_ref, ow_ref, ob_ref, al_ref, be_ref,
                    out_ref):
    f32 = jnp.float32
    bf16 = jnp.bfloat16
    hw = x_ref.shape[1]

    def mm(a, b, dims=((1,), (0,))):
        return jax.lax.dot_general(a.astype(bf16), b.astype(bf16),
                                   (dims, ((), ())),
                                   preferred_element_type=f32)

    ri = jax.lax.broadcasted_iota(jnp.int32, (DIM, DIM), 0)
    ci = jax.lax.broadcasted_iota(jnp.int32, (DIM, DIM), 1)
    bdmask = (ri // HDIM) == (ci // HDIM)
    bd_ones = jnp.where(bdmask, 1.0, 0.0)
    bd_lower = jnp.where(bdmask & (ri > ci), 1.0, 0.0)

    xb = x_ref[0]
    xp = mm(xb, pw_ref[...], ((1,), (1,))) + pb_ref[...]

    x3 = xp.reshape(hw // 4, 4, PDIM)
    m1 = jnp.maximum(jnp.maximum(x3[:, 0], x3[:, 1]),
                     jnp.maximum(x3[:, 2], x3[:, 3]))
    m4 = m1.reshape(NANCH, 4, NANCH, PDIM)
    m2 = jnp.maximum(jnp.maximum(m4[:, 0], m4[:, 1]),
                     jnp.maximum(m4[:, 2], m4[:, 3]))
    pooled = m2.reshape(A, PDIM)

    P0, V0 = xp[:, :DIM], xp[:, DIM:]
    P1, V1 = pooled[:, :DIM], pooled[:, DIM:]

    sq = P0 * P0
    hi = sq.astype(bf16)
    mid = (sq - hi.astype(f32)).astype(bf16)
    lo = ((sq - hi.astype(f32)) - mid.astype(f32)).astype(bf16)
    bd16 = bd_ones.astype(bf16)
    dd = (((1,), (0,)), ((), ()))
    ss = (jax.lax.dot_general(hi, bd16, dd, preferred_element_type=f32)
          + jax.lax.dot_general(mid, bd16, dd, preferred_element_type=f32)
          + jax.lax.dot_general(lo, bd16, dd, preferred_element_type=f32))
    P0n = P0 * jax.lax.rsqrt(jnp.maximum(ss, 1e-24))

    p1n = []
    for h in range(NHEADS):
        p1h = P1[:, h * HDIM:(h + 1) * HDIM]
        n1 = jnp.sqrt(jnp.sum(p1h * p1h, axis=1, keepdims=True))
        p1n.append(p1h / jnp.maximum(n1, 1e-12))
    P1n = jnp.concatenate(p1n, axis=1)

    P1BD = jnp.where(bdmask, jnp.concatenate([P1n.T] * NHEADS, axis=1), 0.0)
    sim = mm(P0n, P1BD)
    sim = jax.nn.sigmoid(al_ref[0, 0] * sim + be_ref[0, 0])

    eqs = []
    for h in range(NHEADS):
        sh = sim[:, h * A:(h + 1) * A]
        mx = jnp.max(sh, axis=1, keepdims=True)
        eqs.append(jnp.where(sh >= mx, 1.0, 0.0))
    eqf = jnp.concatenate(eqs, axis=1)
    nearlier = mm(eqf, bd_lower)
    S = jnp.where((eqf > 0.5) & (nearlier < 0.5), sim, 0.0)

    denom = jnp.sum(S, axis=0, keepdims=True) + 1.0

    M1 = mm(S, V0, ((0,), (0,)))
    MSG = jnp.where(bdmask, M1 + jnp.concatenate([V1] * NHEADS, axis=0), 0.0)
    MSG = MSG / denom.T

    out_tok = mm(S, MSG)
    y = mm(out_tok, ow_ref[...], ((1,), (1,))) + ob_ref[...]
    out_ref[0] = y


def kernel(x, proj_w, proj_b, out_w, out_b, alpha, beta):
    n, h, w, _ = x.shape
    hw = h * w
    pw = proj_w[_PERM]
    pb = proj_b[_PERM].reshape(1, PDIM)
    out = pl.pallas_call(
        _cluster_kernel,
        grid=(n,),
        in_specs=[
            pl.BlockSpec((1, hw, DIM), lambda i: (i, 0, 0)),
            pl.BlockSpec((PDIM, DIM), lambda i: (0, 0)),
            pl.BlockSpec((1, PDIM), lambda i: (0, 0)),
            pl.BlockSpec((DIM, DIM), lambda i: (0, 0)),
            pl.BlockSpec((1, DIM), lambda i: (0, 0)),
            pl.BlockSpec((1, 1), lambda i: (0, 0)),
            pl.BlockSpec((1, 1), lambda i: (0, 0)),
        ],
        out_specs=pl.BlockSpec((1, hw, DIM), lambda i: (i, 0, 0)),
        out_shape=jax.ShapeDtypeStruct((n, hw, DIM), jnp.float32),
    )(x.reshape(n, hw, DIM), pw, pb, out_w, out_b.reshape(1, DIM),
      alpha.reshape(1, 1), beta.reshape(1, 1))
    return out.reshape(n, h, w, DIM)


# speedup vs baseline: 4.2147x; 1.1595x over previous
import jax
import jax.numpy as jnp
from jax.experimental import pallas as pl

DIM = 384
NHEADS = 6
HDIM = DIM // NHEADS
NANCH = 8
A = NANCH * NANCH
PDIM = 2 * DIM


def _cluster_kernel(x_ref, pw_ref, pb_ref, ow_ref, ob_ref, al_ref, be_ref,
                    out_ref):
    f32 = jnp.float32
    bf16 = jnp.bfloat16
    hw = x_ref.shape[1]
    dd = (((1,), (0,)), ((), ()))

    def mm(a, b, dims=((1,), (0,))):
        return jax.lax.dot_general(a, b, (dims, ((), ())),
                                   preferred_element_type=f32)

    ri = jax.lax.broadcasted_iota(jnp.int32, (DIM, DIM), 0)
    ci = jax.lax.broadcasted_iota(jnp.int32, (DIM, DIM), 1)
    bdmask = (ri // HDIM) == (ci // HDIM)
    bd_lower = jnp.where(bdmask & (ri > ci), 1.0, 0.0).astype(bf16)

    pw = pw_ref[...]
    pwp = jnp.concatenate(
        [pw[2 * h * HDIM:(2 * h + 1) * HDIM] for h in range(NHEADS)]
        + [pw[(2 * h + 1) * HDIM:(2 * h + 2) * HDIM] for h in range(NHEADS)],
        axis=0).astype(bf16)
    pb = pb_ref[...]
    pbp = jnp.concatenate(
        [pb[:, 2 * h * HDIM:(2 * h + 1) * HDIM] for h in range(NHEADS)]
        + [pb[:, (2 * h + 1) * HDIM:(2 * h + 2) * HDIM]
           for h in range(NHEADS)],
        axis=1)

    xb = x_ref[0]
    xp = mm(xb.astype(bf16), pwp, ((1,), (1,))) + pbp

    x3 = xp.reshape(hw // 4, 4, PDIM)
    m1 = jnp.maximum(jnp.maximum(x3[:, 0], x3[:, 1]),
                     jnp.maximum(x3[:, 2], x3[:, 3]))
    m4 = m1.reshape(NANCH, 4, NANCH, PDIM)
    m2 = jnp.maximum(jnp.maximum(m4[:, 0], m4[:, 1]),
                     jnp.maximum(m4[:, 2], m4[:, 3]))
    pooled = m2.reshape(A, PDIM)

    P0, V0 = xp[:, :DIM], xp[:, DIM:]
    P1, V1 = pooled[:, :DIM], pooled[:, DIM:]

    sq = P0 * P0
    hi = sq.astype(bf16)
    r1 = sq - hi.astype(f32)
    mid = r1.astype(bf16)
    lo = (r1 - mid.astype(f32)).astype(bf16)
    cat = jnp.concatenate([hi, mid, lo], axis=1)
    r3 = jax.lax.broadcasted_iota(jnp.int32, (3 * DIM, DIM), 0)
    c3 = jax.lax.broadcasted_iota(jnp.int32, (3 * DIM, DIM), 1)
    bd3 = jnp.where((r3 % DIM) // HDIM == c3 // HDIM, 1.0, 0.0).astype(bf16)
    ss = jax.lax.dot_general(cat, bd3, dd, preferred_element_type=f32)
    P0n = P0 * jax.lax.rsqrt(jnp.maximum(ss, 1e-24))

    p1n = []
    for h in range(NHEADS):
        p1h = P1[:, h * HDIM:(h + 1) * HDIM]
        n1 = jnp.sqrt(jnp.sum(p1h * p1h, axis=1, keepdims=True))
        p1n.append(p1h / jnp.maximum(n1, 1e-12))
    P1n = jnp.concatenate(p1n, axis=1)

    P1BD = jnp.where(bdmask, jnp.concatenate([P1n.T] * NHEADS, axis=1),
                     0.0).astype(bf16)
    sim = mm(P0n.astype(bf16), P1BD)
    sim = jax.nn.sigmoid(al_ref[0, 0] * sim + be_ref[0, 0])

    eqs = []
    for h in range(NHEADS):
        sh = sim[:, h * A:(h + 1) * A]
        mx = jnp.max(sh, axis=1, keepdims=True)
        eqs.append(jnp.where(sh >= mx, 1.0, 0.0).astype(bf16))
    eqf = jnp.concatenate(eqs, axis=1)
    nearlier = mm(eqf, bd_lower)
    half = jnp.bfloat16(0.5)
    S = jnp.where((eqf > half) & (nearlier < 0.5), sim, 0.0)

    denom = jnp.sum(S, axis=0, keepdims=True) + 1.0
    Sb = S.astype(bf16)

    M1 = mm(Sb, V0.astype(bf16), ((0,), (0,)))
    MSG = jnp.where(bdmask, M1 + jnp.concatenate([V1] * NHEADS, axis=0), 0.0)
    MSG = MSG / denom.T

    out_tok = mm(Sb, MSG.astype(bf16))
    y = mm(out_tok.astype(bf16), ow_ref[...].astype(bf16), ((1,), (1,)))
    out_ref[0] = y + ob_ref[...]


def kernel(x, proj_w, proj_b, out_w, out_b, alpha, beta):
    n, h, w, _ = x.shape
    hw = h * w
    out = pl.pallas_call(
        _cluster_kernel,
        grid=(n,),
        in_specs=[
            pl.BlockSpec((1, hw, DIM), lambda i: (i, 0, 0)),
            pl.BlockSpec((PDIM, DIM), lambda i: (0, 0)),
            pl.BlockSpec((1, PDIM), lambda i: (0, 0)),
            pl.BlockSpec((DIM, DIM), lambda i: (0, 0)),
            pl.BlockSpec((1, DIM), lambda i: (0, 0)),
            pl.BlockSpec((1, 1), lambda i: (0, 0)),
            pl.BlockSpec((1, 1), lambda i: (0, 0)),
        ],
        out_specs=pl.BlockSpec((1, hw, DIM), lambda i: (i, 0, 0)),
        out_shape=jax.ShapeDtypeStruct((n, hw, DIM), jnp.float32),
    )(x.reshape(n, hw, DIM), proj_w, proj_b.reshape(1, PDIM), out_w,
      out_b.reshape(1, DIM), alpha.reshape(1, 1), beta.reshape(1, 1))
    return out.reshape(n, h, w, DIM)


# speedup vs baseline: 4.4060x; 1.0454x over previous
import numpy as np
import jax
import jax.numpy as jnp
from jax.experimental import pallas as pl
from jax.experimental.pallas import tpu as pltpu

DIM = 384
NHEADS = 6
HDIM = DIM // NHEADS
NANCH = 8
A = NANCH * NANCH
PDIM = 2 * DIM

_R = np.arange(DIM)[:, None]
_C = np.arange(DIM)[None, :]
_BDMASK = ((_R // HDIM) == (_C // HDIM)).astype(np.float32)
_BD_LOWER = (_BDMASK * (_R > _C)).astype(np.float32)
_R3 = np.arange(3 * DIM)[:, None] % DIM
_BD3 = ((_R3 // HDIM) == (_C // HDIM)).astype(np.float32)


def _cluster_kernel(x_ref, pw_ref, pb_ref, ow_ref, ob_ref, al_ref, be_ref,
                    bdm_ref, bdl_ref, bd3_ref, out_ref, pwp_ref, pbp_ref):
    f32 = jnp.float32
    bf16 = jnp.bfloat16
    hw = x_ref.shape[1]
    dd = (((1,), (0,)), ((), ()))

    def mm(a, b, dims=((1,), (0,))):
        return jax.lax.dot_general(a, b, (dims, ((), ())),
                                   preferred_element_type=f32)

    @pl.when(pl.program_id(0) == 0)
    def _():
        pw = pw_ref[...]
        pwp_ref[...] = jnp.concatenate(
            [pw[2 * h * HDIM:(2 * h + 1) * HDIM] for h in range(NHEADS)]
            + [pw[(2 * h + 1) * HDIM:(2 * h + 2) * HDIM]
               for h in range(NHEADS)],
            axis=0).astype(bf16)
        pb = pb_ref[...]
        pbp_ref[...] = jnp.concatenate(
            [pb[:, 2 * h * HDIM:(2 * h + 1) * HDIM] for h in range(NHEADS)]
            + [pb[:, (2 * h + 1) * HDIM:(2 * h + 2) * HDIM]
               for h in range(NHEADS)],
            axis=1)

    bdmask = bdm_ref[...] > 0.5

    xb = x_ref[0]
    xp = mm(xb.astype(bf16), pwp_ref[...], ((1,), (1,))) + pbp_ref[...]

    x3 = xp.reshape(hw // 4, 4, PDIM)
    m1 = jnp.maximum(jnp.maximum(x3[:, 0], x3[:, 1]),
                     jnp.maximum(x3[:, 2], x3[:, 3]))
    m4 = m1.reshape(NANCH, 4, NANCH, PDIM)
    m2 = jnp.maximum(jnp.maximum(m4[:, 0], m4[:, 1]),
                     jnp.maximum(m4[:, 2], m4[:, 3]))
    pooled = m2.reshape(A, PDIM)

    P0, V0 = xp[:, :DIM], xp[:, DIM:]
    P1, V1 = pooled[:, :DIM], pooled[:, DIM:]

    sq = P0 * P0
    hi = sq.astype(bf16)
    r1 = sq - hi.astype(f32)
    mid = r1.astype(bf16)
    lo = (r1 - mid.astype(f32)).astype(bf16)
    cat = jnp.concatenate([hi, mid, lo], axis=1)
    ss = jax.lax.dot_general(cat, bd3_ref[...], dd,
                             preferred_element_type=f32)
    P0n = P0 * jax.lax.rsqrt(jnp.maximum(ss, 1e-24))

    p1n = []
    for h in range(NHEADS):
        p1h = P1[:, h * HDIM:(h + 1) * HDIM]
        n1 = jnp.sqrt(jnp.sum(p1h * p1h, axis=1, keepdims=True))
        p1n.append(p1h / jnp.maximum(n1, 1e-12))
    P1n = jnp.concatenate(p1n, axis=1)

    P1BD = jnp.where(bdmask, jnp.concatenate([P1n.T] * NHEADS, axis=1),
                     0.0).astype(bf16)
    sim = mm(P0n.astype(bf16), P1BD)
    sim = jax.nn.sigmoid(al_ref[0, 0] * sim + be_ref[0, 0])

    eqs = []
    for h in range(NHEADS):
        sh = sim[:, h * A:(h + 1) * A]
        mx = jnp.max(sh, axis=1, keepdims=True)
        eqs.append(jnp.where(sh >= mx, 1.0, 0.0).astype(bf16))
    eqf = jnp.concatenate(eqs, axis=1)
    nearlier = mm(eqf, bdl_ref[...])
    half = jnp.bfloat16(0.5)
    Sb = jnp.where((eqf > half) & (nearlier < 0.5), sim.astype(bf16),
                   jnp.bfloat16(0.0))

    oc_iota = jax.lax.broadcasted_iota(jnp.int32, (hw, 2 * HDIM), 1)
    onescol = jnp.where(oc_iota == 0, 1.0, 0.0).astype(bf16)
    V0e = jnp.concatenate([V0.astype(bf16), onescol], axis=1)
    M1e = mm(Sb, V0e, ((0,), (0,)))
    denomc = M1e[:, DIM:DIM + 1] + 1.0
    MSG = jnp.where(bdmask,
                    M1e[:, :DIM] + jnp.concatenate([V1] * NHEADS, axis=0),
                    0.0)
    MSG = MSG / denomc

    out_tok = mm(Sb, MSG.astype(bf16))
    y = mm(out_tok.astype(bf16), ow_ref[...].astype(bf16), ((1,), (1,)))
    out_ref[0] = y + ob_ref[...]


def kernel(x, proj_w, proj_b, out_w, out_b, alpha, beta):
    n, h, w, _ = x.shape
    hw = h * w
    full = lambda i: (0, 0)
    out = pl.pallas_call(
        _cluster_kernel,
        grid=(n,),
        in_specs=[
            pl.BlockSpec((1, hw, DIM), lambda i: (i, 0, 0)),
            pl.BlockSpec((PDIM, DIM), full),
            pl.BlockSpec((1, PDIM), full),
            pl.BlockSpec((DIM, DIM), full),
            pl.BlockSpec((1, DIM), full),
            pl.BlockSpec((1, 1), full),
            pl.BlockSpec((1, 1), full),
            pl.BlockSpec((DIM, DIM), full),
            pl.BlockSpec((DIM, DIM), full),
            pl.BlockSpec((3 * DIM, DIM), full),
        ],
        out_specs=pl.BlockSpec((1, hw, DIM), lambda i: (i, 0, 0)),
        out_shape=jax.ShapeDtypeStruct((n, hw, DIM), jnp.float32),
        scratch_shapes=[
            pltpu.VMEM((PDIM, DIM), jnp.bfloat16),
            pltpu.VMEM((1, PDIM), jnp.float32),
        ],
    )(x.reshape(n, hw, DIM), proj_w, proj_b.reshape(1, PDIM), out_w,
      out_b.reshape(1, DIM), alpha.reshape(1, 1), beta.reshape(1, 1),
      jnp.asarray(_BDMASK),
      jnp.asarray(_BD_LOWER, dtype=jnp.bfloat16),
      jnp.asarray(_BD3, dtype=jnp.bfloat16))
    return out.reshape(n, h, w, DIM)


# speedup vs baseline: 4.5078x; 1.0231x over previous
import numpy as np
import jax
import jax.numpy as jnp
from jax.experimental import pallas as pl
from jax.experimental.pallas import tpu as pltpu

DIM = 384
NHEADS = 6
HDIM = DIM // NHEADS
NANCH = 8
A = NANCH * NANCH
PDIM = 2 * DIM

_R = np.arange(DIM)[:, None]
_C = np.arange(DIM)[None, :]
_BDMASK = ((_R // HDIM) == (_C // HDIM)).astype(np.float32)
_BD_LOWER = (_BDMASK * (_R > _C)).astype(np.float32)
_R3 = np.arange(3 * DIM)[:, None] % DIM
_BD3 = ((_R3 // HDIM) == (_C // HDIM)).astype(np.float32)


def _cluster_kernel(x_ref, pw_ref, pb_ref, ow_ref, ob_ref, al_ref, be_ref,
                    bdm_ref, bdl_ref, bd3_ref, out_ref, pwp_ref, pbp_ref):
    f32 = jnp.float32
    bf16 = jnp.bfloat16
    hw = x_ref.shape[1]
    dd = (((1,), (0,)), ((), ()))

    def mm(a, b, dims=((1,), (0,))):
        return jax.lax.dot_general(a, b, (dims, ((), ())),
                                   preferred_element_type=f32)

    @pl.when(pl.program_id(0) == 0)
    def _():
        pw = pw_ref[...]
        pwp_ref[...] = jnp.concatenate(
            [pw[2 * h * HDIM:(2 * h + 1) * HDIM] for h in range(NHEADS)]
            + [pw[(2 * h + 1) * HDIM:(2 * h + 2) * HDIM]
               for h in range(NHEADS)],
            axis=0).astype(bf16)
        pb = pb_ref[...]
        pbp_ref[...] = jnp.concatenate(
            [pb[:, 2 * h * HDIM:(2 * h + 1) * HDIM] for h in range(NHEADS)]
            + [pb[:, (2 * h + 1) * HDIM:(2 * h + 2) * HDIM]
               for h in range(NHEADS)],
            axis=1)

    bdmask = bdm_ref[...] > 0.5

    xb = x_ref[0]
    xp = mm(xb.astype(bf16), pwp_ref[...], ((1,), (1,))) + pbp_ref[...]

    x3 = xp.reshape(hw // 4, 4, PDIM)
    m1 = jnp.maximum(jnp.maximum(x3[:, 0], x3[:, 1]),
                     jnp.maximum(x3[:, 2], x3[:, 3]))
    m4 = m1.reshape(NANCH, 4, NANCH, PDIM)
    m2 = jnp.maximum(jnp.maximum(m4[:, 0], m4[:, 1]),
                     jnp.maximum(m4[:, 2], m4[:, 3]))
    pooled = m2.reshape(A, PDIM)

    P0, V0 = xp[:, :DIM], xp[:, DIM:]
    P1, V1 = pooled[:, :DIM], pooled[:, DIM:]

    sq = P0 * P0
    hi = sq.astype(bf16)
    r1 = sq - hi.astype(f32)
    mid = r1.astype(bf16)
    lo = (r1 - mid.astype(f32)).astype(bf16)
    cat = jnp.concatenate([hi, mid, lo], axis=1)
    ss = jax.lax.dot_general(cat, bd3_ref[...], dd,
                             preferred_element_type=f32)
    P0n = P0 * jax.lax.rsqrt(jnp.maximum(ss, 1e-24))

    p1n = []
    for h in range(NHEADS):
        p1h = P1[:, h * HDIM:(h + 1) * HDIM]
        n1 = jnp.sqrt(jnp.sum(p1h * p1h, axis=1, keepdims=True))
        p1n.append(p1h / jnp.maximum(n1, 1e-12))
    P1n = jnp.concatenate(p1n, axis=1)

    P1BD = jnp.where(bdmask, jnp.concatenate([P1n.T] * NHEADS, axis=1),
                     0.0).astype(bf16)
    sim = mm(P0n.astype(bf16), P1BD)
    sim = jax.nn.sigmoid(al_ref[0, 0] * sim + be_ref[0, 0])

    eqs = []
    for h in range(NHEADS):
        sh = sim[:, h * A:(h + 1) * A]
        mx = jnp.max(sh, axis=1, keepdims=True)
        eqs.append(jnp.where(sh >= mx, 1.0, 0.0).astype(bf16))
    eqf = jnp.concatenate(eqs, axis=1)
    nearlier = mm(eqf, bdl_ref[...])
    half = jnp.bfloat16(0.5)
    Sb = jnp.where((eqf > half) & (nearlier < 0.5), sim.astype(bf16),
                   jnp.bfloat16(0.0))

    oc_iota = jax.lax.broadcasted_iota(jnp.int32, (hw, 2 * HDIM), 1)
    onescol = jnp.where(oc_iota == 0, 1.0, 0.0).astype(bf16)
    V0e = jnp.concatenate([V0.astype(bf16), onescol], axis=1)
    M1e = mm(Sb, V0e, ((0,), (0,)))
    denomc = M1e[:, DIM:DIM + 1] + 1.0
    MSG = jnp.where(bdmask,
                    M1e[:, :DIM] + jnp.concatenate([V1] * NHEADS, axis=0),
                    0.0)
    MSG = MSG / denomc

    W2 = mm(MSG.astype(bf16), ow_ref[...].astype(bf16), ((1,), (1,)))
    y = mm(Sb, W2.astype(bf16))
    out_ref[0] = y + ob_ref[...]


def kernel(x, proj_w, proj_b, out_w, out_b, alpha, beta):
    n, h, w, _ = x.shape
    hw = h * w
    full = lambda i: (0, 0)
    out = pl.pallas_call(
        _cluster_kernel,
        grid=(n,),
        in_specs=[
            pl.BlockSpec((1, hw, DIM), lambda i: (i, 0, 0)),
            pl.BlockSpec((PDIM, DIM), full),
            pl.BlockSpec((1, PDIM), full),
            pl.BlockSpec((DIM, DIM), full),
            pl.BlockSpec((1, DIM), full),
            pl.BlockSpec((1, 1), full),
            pl.BlockSpec((1, 1), full),
            pl.BlockSpec((DIM, DIM), full),
            pl.BlockSpec((DIM, DIM), full),
            pl.BlockSpec((3 * DIM, DIM), full),
        ],
        out_specs=pl.BlockSpec((1, hw, DIM), lambda i: (i, 0, 0)),
        out_shape=jax.ShapeDtypeStruct((n, hw, DIM), jnp.float32),
        scratch_shapes=[
            pltpu.VMEM((PDIM, DIM), jnp.bfloat16),
            pltpu.VMEM((1, PDIM), jnp.float32),
        ],
    )(x.reshape(n, hw, DIM), proj_w, proj_b.reshape(1, PDIM), out_w,
      out_b.reshape(1, DIM), alpha.reshape(1, 1), beta.reshape(1, 1),
      jnp.asarray(_BDMASK),
      jnp.asarray(_BD_LOWER, dtype=jnp.bfloat16),
      jnp.asarray(_BD3, dtype=jnp.bfloat16))
    return out.reshape(n, h, w, DIM)
